# baseline (speedup 1.0000x reference)
import dataclasses
import functools

import jax
import jax.numpy as jnp
from jax import lax
from jax.experimental import pallas as pl
from jax.experimental.pallas import tpu as pltpu
from jax.experimental.pallas import tpu_sc as plsc

_NC = 2
_NS = 16
_NW = _NC * _NS
_K = 40
_BR = 2000


def _mesh():
    return plsc.VectorSubcoreMesh(
        core_axis_name="c", subcore_axis_name="s",
        num_cores=_NC, num_subcores=_NS)


def _deg_kernel(n_acc, m):
    cp = pltpu.CompilerParams()
    if "needs_layout_passes" in pltpu.CompilerParams.__dataclass_fields__:
        cp = dataclasses.replace(cp, needs_layout_passes=False)

    @functools.partial(
        pl.kernel,
        out_type=jax.ShapeDtypeStruct((_NW, n_acc), jnp.float32),
        mesh=_mesh(),
        compiler_params=cp,
        scratch_types=[
            pltpu.VMEM((m, 64), jnp.int32),
            pltpu.VMEM((n_acc,), jnp.float32),
        ],
    )
    def deg(dsti_hbm, out_hbm, dstv, hist):
        cid = lax.axis_index("c")
        sid = lax.axis_index("s")
        wid = cid * _NS + sid
        pltpu.sync_copy(dsti_hbm.at[wid], dstv)

        @pl.loop(0, n_acc // 16)
        def _(i):
            hist.at[pl.ds(i * 16, 16)][...] = jnp.zeros((16,), jnp.float32)

        ones = jnp.ones((16,), jnp.float32)

        @pl.loop(0, m)
        def _(j):
            for b in range(4):
                idxv = dstv[j, pl.ds(b * 16, 16)]
                plsc.addupdate_scatter(hist, [idxv], ones)

        pltpu.sync_copy(hist, out_hbm.at[wid])

    return deg


_G = 16
_NB = 8


def _agg_kernel(n_acc, d, chunks):
    rpw = n_acc // _NS
    ngrp = chunks // _G
    assert chunks % (2 * _G) == 0 and _G % _NB == 0

    @functools.partial(
        pl.kernel,
        out_type=jax.ShapeDtypeStruct((_NC, n_acc, d), jnp.float32),
        mesh=_mesh(),
        scratch_types=[
            pltpu.VMEM((_G, _K), jnp.int32),
            pltpu.VMEM((_G, _K), jnp.int32),
            pltpu.VMEM((_G, _K), jnp.int32),
            pltpu.VMEM((_G, _K), jnp.int32),
            [pltpu.VMEM((_K, d), jnp.float32) for _ in range(_NB)],
            [pltpu.SemaphoreType.DMA for _ in range(_NB)],
            [pltpu.SemaphoreType.DMA for _ in range(_NB)],
            pltpu.SemaphoreType.DMA,
            pltpu.SemaphoreType.DMA,
            pltpu.VMEM_SHARED((n_acc, d), jnp.float32),
        ],
    )
    def agg(g_hbm, srci_hbm, dsti_hbm, out_hbm, sg0, dg0, sg1, dg1,
            bufs, gsem, ssem, isem, csem, acc):
        cid = lax.axis_index("c")
        sid = lax.axis_index("s")
        wid = cid * _NS + sid

        pltpu.sync_copy(srci_hbm.at[wid, pl.ds(0, _G)], sg0)
        pltpu.sync_copy(dsti_hbm.at[wid, pl.ds(0, _G)], dg0)

        @pl.loop(0, _K)
        def _(r):
            for cc in range(d // 16):
                bufs[0].at[pl.ds(r, 1), pl.ds(cc * 16, 16)][...] = (
                    jnp.zeros((1, 16), jnp.float32))

        row0 = sid * rpw
        nslice = rpw // _K
        for j in range(nslice):
            pltpu.async_copy(bufs[0], acc.at[pl.ds(row0 + j * _K, _K)], csem)
        for j in range(nslice):
            pltpu.make_async_copy(
                bufs[0], acc.at[pl.ds(row0 + j * _K, _K)], csem).wait()

        plsc.subcore_barrier()

        for b in range(_NB - 1):
            pltpu.async_copy(g_hbm.at[sg0.at[b]], bufs[b], gsem[b])

        def do_group(sg, dg, sgn, dgn, gg):
            @pl.when(gg + 1 < ngrp)
            def _():
                nxt = (gg + 1) * _G
                pltpu.async_copy(srci_hbm.at[wid, pl.ds(nxt, _G)], sgn, isem)
                pltpu.async_copy(dsti_hbm.at[wid, pl.ds(nxt, _G)], dgn, isem)

            ahead = _NB - 1
            for b in range(_G):
                bi = b % _NB
                bj = (b + ahead) % _NB
                pltpu.make_async_copy(g_hbm.at[sg.at[b]], bufs[bi],
                                      gsem[bi]).wait()
                if b == _G - ahead:
                    @pl.when(gg + 1 < ngrp)
                    def _():
                        nxt = (gg + 1) * _G
                        pltpu.make_async_copy(
                            srci_hbm.at[wid, pl.ds(nxt, _G)], sgn, isem).wait()
                        pltpu.make_async_copy(
                            dsti_hbm.at[wid, pl.ds(nxt, _G)], dgn, isem).wait()

                @pl.when(gg * _G + b > 0)
                def _():
                    pltpu.make_async_copy(bufs[bj], acc.at[dg.at[b]],
                                          ssem[bj]).wait()
                if b + ahead < _G:
                    pltpu.async_copy(g_hbm.at[sg.at[b + ahead]], bufs[bj],
                                     gsem[bj])
                else:
                    @pl.when(gg + 1 < ngrp)
                    def _():
                        pltpu.async_copy(g_hbm.at[sgn.at[b + ahead - _G]],
                                         bufs[bj], gsem[bj])
                pltpu.async_copy(bufs[bi], acc.at[dg.at[b]], ssem[bi],
                                 add=True)

        @pl.loop(0, ngrp // 2)
        def _(hh):
            do_group(sg0, dg0, sg1, dg1, hh * 2)
            do_group(sg1, dg1, sg0, dg0, hh * 2 + 1)

        pltpu.make_async_copy(bufs[(chunks - 1) % _NB], acc.at[dg1.at[_G - 1]],
                              ssem[(chunks - 1) % _NB]).wait()

        plsc.subcore_barrier()

        for j in range(nslice):
            sl = pl.ds(row0 + j * _K, _K)
            pltpu.async_copy(acc.at[sl], out_hbm.at[cid, sl], csem)
        for j in range(nslice):
            sl = pl.ds(row0 + j * _K, _K)
            pltpu.make_async_copy(acc.at[sl], out_hbm.at[cid, sl], csem).wait()

    return agg


def _first_tc(n, n_acc, d_in, hid):
    grid = n // _BR

    def body(deg_ref, x_ref, w_ref, dinv_ref, g_ref):
        total = jnp.sum(deg_ref[...], axis=1, keepdims=True) + 1.0
        dinv = lax.rsqrt(total)
        hw = jnp.dot(x_ref[...], w_ref[...],
                     preferred_element_type=jnp.float32,
                     precision=lax.Precision.HIGHEST)
        dinv_ref[...] = dinv
        g_ref[...] = hw * dinv

    return pl.pallas_call(
        body,
        grid=(grid,),
        in_specs=[
            pl.BlockSpec((_BR, _NW), lambda i: (i, 0)),
            pl.BlockSpec((_BR, d_in), lambda i: (i, 0)),
            pl.BlockSpec((d_in, hid), lambda i: (0, 0)),
        ],
        out_specs=[
            pl.BlockSpec((_BR, 1), lambda i: (i, 0)),
            pl.BlockSpec((_BR, hid), lambda i: (i, 0)),
        ],
        out_shape=[
            jax.ShapeDtypeStruct((n, 1), jnp.float32),
            jax.ShapeDtypeStruct((n_acc, hid), jnp.float32),
        ],
    )


def _mid_tc(n, n_acc, d, d_next, d_store):
    grid = n // _BR

    def body(agg_ref, g_ref, dinv_ref, b_ref, w_ref, out_ref):
        s = (agg_ref[0] + agg_ref[1] + g_ref[...]) * dinv_ref[...] + b_ref[...]
        h = jnp.maximum(s, 0.0)
        w = w_ref[...]
        if d_store > d_next:
            w = jnp.concatenate(
                [w, jnp.zeros((d, d_store - d_next), jnp.float32)], axis=1)
        out_ref[...] = jnp.dot(h, w,
                               preferred_element_type=jnp.float32,
                               precision=lax.Precision.HIGHEST) * dinv_ref[...]

    return pl.pallas_call(
        body,
        grid=(grid,),
        in_specs=[
            pl.BlockSpec((_NC, _BR, d), lambda i: (0, i, 0)),
            pl.BlockSpec((_BR, d), lambda i: (i, 0)),
            pl.BlockSpec((_BR, 1), lambda i: (i, 0)),
            pl.BlockSpec((1, d), lambda i: (0, 0)),
            pl.BlockSpec((d, d_next), lambda i: (0, 0)),
        ],
        out_specs=pl.BlockSpec((_BR, d_store), lambda i: (i, 0)),
        out_shape=jax.ShapeDtypeStruct((n_acc, d_store), jnp.float32),
    )


def _final_tc(n, d, d_store):
    grid = n // _BR

    def body(agg_ref, g_ref, dinv_ref, b_ref, out_ref):
        agg = agg_ref[0, :, :d] + agg_ref[1, :, :d] + g_ref[:, :d]
        out_ref[...] = agg * dinv_ref[...] + b_ref[...]

    return pl.pallas_call(
        body,
        grid=(grid,),
        in_specs=[
            pl.BlockSpec((_NC, _BR, d_store), lambda i: (0, i, 0)),
            pl.BlockSpec((_BR, d_store), lambda i: (i, 0)),
            pl.BlockSpec((_BR, 1), lambda i: (i, 0)),
            pl.BlockSpec((1, d), lambda i: (0, 0)),
        ],
        out_specs=pl.BlockSpec((_BR, d), lambda i: (i, 0)),
        out_shape=jax.ShapeDtypeStruct((n, d), jnp.float32),
    )


def kernel(x, edge_index, W1, b1, W2, b2, W3, b3):
    n, d_in = x.shape
    hid = W1.shape[1]
    c_out = W3.shape[1]
    e = edge_index.shape[1]

    ew = _NW * _K
    gsz = 2 * _G
    chunks = ((-(-e // ew)) + gsz - 1) // gsz * gsz
    e_pad = chunks * ew
    slab = _NS * _K
    n_acc = -(-(n + 1) // slab) * slab

    pad = e_pad - e
    fill = n + jnp.arange(pad, dtype=jnp.int32) % (n_acc - n)
    srcp = jnp.concatenate([edge_index[0], fill]).reshape(_NW, chunks, _K)
    dstp = jnp.concatenate([edge_index[1], fill]).reshape(_NW, chunks, _K)

    c_store = -(-c_out // 128) * 128

    m64 = chunks * _K // 64
    degp = _deg_kernel(n_acc, m64)(dstp.reshape(_NW, m64, 64))
    dinv, g1 = _first_tc(n, n_acc, d_in, hid)(degp.T, x, W1)
    agg1 = _agg_kernel(n_acc, hid, chunks)(g1, srcp, dstp)
    g2 = _mid_tc(n, n_acc, hid, hid, hid)(agg1, g1, dinv, b1.reshape(1, -1), W2)
    agg2 = _agg_kernel(n_acc, hid, chunks)(g2, srcp, dstp)
    g3 = _mid_tc(n, n_acc, hid, c_out, c_store)(agg2, g2, dinv,
                                                b2.reshape(1, -1), W3)
    agg3 = _agg_kernel(n_acc, c_store, chunks)(g3, srcp, dstp)
    return _final_tc(n, c_out, c_store)(agg3, g3, dinv, b3.reshape(1, -1))


# speedup vs baseline: 1.0097x; 1.0097x over previous
import dataclasses
import functools

import jax
import jax.numpy as jnp
from jax import lax
from jax.experimental import pallas as pl
from jax.experimental.pallas import tpu as pltpu
from jax.experimental.pallas import tpu_sc as plsc

_NC = 2
_NS = 16
_NW = _NC * _NS
_K = 80
_BR = 2000


def _mesh():
    return plsc.VectorSubcoreMesh(
        core_axis_name="c", subcore_axis_name="s",
        num_cores=_NC, num_subcores=_NS)


def _deg_kernel(n_acc, chunks):
    cp = pltpu.CompilerParams()
    if "needs_layout_passes" in pltpu.CompilerParams.__dataclass_fields__:
        cp = dataclasses.replace(cp, needs_layout_passes=False)

    @functools.partial(
        pl.kernel,
        out_type=jax.ShapeDtypeStruct((_NW, n_acc), jnp.float32),
        mesh=_mesh(),
        compiler_params=cp,
        scratch_types=[
            pltpu.VMEM((chunks, _K), jnp.int32),
            pltpu.VMEM((n_acc,), jnp.float32),
        ],
    )
    def deg(dsti_hbm, out_hbm, dstv, hist):
        cid = lax.axis_index("c")
        sid = lax.axis_index("s")
        wid = cid * _NS + sid
        pltpu.sync_copy(dsti_hbm.at[wid], dstv)

        @pl.loop(0, n_acc // 16)
        def _(i):
            hist.at[pl.ds(i * 16, 16)][...] = jnp.zeros((16,), jnp.float32)

        ones = jnp.ones((16,), jnp.float32)

        @pl.loop(0, chunks)
        def _(j):
            for b in range(_K // 16):
                idxv = dstv[j, pl.ds(b * 16, 16)]
                plsc.addupdate_scatter(hist, [idxv], ones)

        pltpu.sync_copy(hist, out_hbm.at[wid])

    return deg


_G = 8


def _agg_kernel(n_acc, d, chunks):
    rpw = n_acc // _NS
    ngrp = chunks // _G
    assert chunks % (2 * _G) == 0

    @functools.partial(
        pl.kernel,
        out_type=jax.ShapeDtypeStruct((_NC, n_acc, d), jnp.float32),
        mesh=_mesh(),
        scratch_types=[
            pltpu.VMEM((_G, _K), jnp.int32),
            pltpu.VMEM((_G, _K), jnp.int32),
            pltpu.VMEM((_G, _K), jnp.int32),
            pltpu.VMEM((_G, _K), jnp.int32),
            pltpu.VMEM((_K, d), jnp.float32),
            pltpu.VMEM((_K, d), jnp.float32),
            pltpu.VMEM((_K, d), jnp.float32),
            pltpu.VMEM((_K, d), jnp.float32),
            pltpu.SemaphoreType.DMA,
            pltpu.SemaphoreType.DMA,
            pltpu.SemaphoreType.DMA,
            pltpu.SemaphoreType.DMA,
            pltpu.SemaphoreType.DMA,
            pltpu.SemaphoreType.DMA,
            pltpu.SemaphoreType.DMA,
            pltpu.SemaphoreType.DMA,
            pltpu.SemaphoreType.DMA,
            pltpu.SemaphoreType.DMA,
            pltpu.VMEM_SHARED((n_acc, d), jnp.float32),
        ],
    )
    def agg(g_hbm, srci_hbm, dsti_hbm, out_hbm, sg0, dg0, sg1, dg1,
            rb0, rb1, rb2, rb3, ga, gb, gc, gd, sa, sb, sc, sd, isem, csem,
            acc):
        bufs = [rb0, rb1, rb2, rb3]
        gsem = [ga, gb, gc, gd]
        ssem = [sa, sb, sc, sd]
        cid = lax.axis_index("c")
        sid = lax.axis_index("s")
        wid = cid * _NS + sid

        pltpu.sync_copy(srci_hbm.at[wid, pl.ds(0, _G)], sg0)
        pltpu.sync_copy(dsti_hbm.at[wid, pl.ds(0, _G)], dg0)

        @pl.loop(0, _K)
        def _(r):
            for cc in range(d // 16):
                bufs[0].at[pl.ds(r, 1), pl.ds(cc * 16, 16)][...] = (
                    jnp.zeros((1, 16), jnp.float32))

        row0 = sid * rpw
        nslice = rpw // _K
        for j in range(nslice):
            pltpu.async_copy(bufs[0], acc.at[pl.ds(row0 + j * _K, _K)], csem)
        for j in range(nslice):
            pltpu.make_async_copy(
                bufs[0], acc.at[pl.ds(row0 + j * _K, _K)], csem).wait()

        plsc.subcore_barrier()

        pltpu.async_copy(g_hbm.at[sg0.at[0]], bufs[0], gsem[0])
        pltpu.async_copy(g_hbm.at[sg0.at[1]], bufs[1], gsem[1])
        pltpu.async_copy(g_hbm.at[sg0.at[2]], bufs[2], gsem[2])

        def do_group(sg, dg, sgn, dgn, gg):
            @pl.when(gg + 1 < ngrp)
            def _():
                nxt = (gg + 1) * _G
                pltpu.async_copy(srci_hbm.at[wid, pl.ds(nxt, _G)], sgn, isem)
                pltpu.async_copy(dsti_hbm.at[wid, pl.ds(nxt, _G)], dgn, isem)

            for b in range(_G):
                bi = b % 4
                bj = (b + 3) % 4
                pltpu.make_async_copy(g_hbm.at[sg.at[b]], bufs[bi],
                                      gsem[bi]).wait()
                if b == _G - 3:
                    @pl.when(gg + 1 < ngrp)
                    def _():
                        nxt = (gg + 1) * _G
                        pltpu.make_async_copy(
                            srci_hbm.at[wid, pl.ds(nxt, _G)], sgn, isem).wait()
                        pltpu.make_async_copy(
                            dsti_hbm.at[wid, pl.ds(nxt, _G)], dgn, isem).wait()

                @pl.when(gg * _G + b > 0)
                def _():
                    pltpu.make_async_copy(bufs[bj], acc.at[dg.at[b]],
                                          ssem[bj]).wait()
                if b + 3 < _G:
                    pltpu.async_copy(g_hbm.at[sg.at[b + 3]], bufs[bj],
                                     gsem[bj])
                else:
                    @pl.when(gg + 1 < ngrp)
                    def _():
                        pltpu.async_copy(g_hbm.at[sgn.at[b + 3 - _G]],
                                         bufs[bj], gsem[bj])
                pltpu.async_copy(bufs[bi], acc.at[dg.at[b]], ssem[bi],
                                 add=True)

        @pl.loop(0, ngrp // 2)
        def _(hh):
            do_group(sg0, dg0, sg1, dg1, hh * 2)
            do_group(sg1, dg1, sg0, dg0, hh * 2 + 1)

        pltpu.make_async_copy(bufs[(chunks - 1) % 4], acc.at[dg1.at[_G - 1]],
                              ssem[(chunks - 1) % 4]).wait()

        plsc.subcore_barrier()

        for j in range(nslice):
            sl = pl.ds(row0 + j * _K, _K)
            pltpu.async_copy(acc.at[sl], out_hbm.at[cid, sl], csem)
        for j in range(nslice):
            sl = pl.ds(row0 + j * _K, _K)
            pltpu.make_async_copy(acc.at[sl], out_hbm.at[cid, sl], csem).wait()

    return agg


def _first_tc(n, n_acc, d_in, hid):
    grid = n // _BR

    def body(deg_ref, x_ref, w_ref, dinv_ref, g_ref):
        total = jnp.sum(deg_ref[...], axis=1, keepdims=True) + 1.0
        dinv = lax.rsqrt(total)
        hw = jnp.dot(x_ref[...], w_ref[...],
                     preferred_element_type=jnp.float32,
                     precision=lax.Precision.HIGHEST)
        dinv_ref[...] = dinv
        g_ref[...] = hw * dinv

    return pl.pallas_call(
        body,
        grid=(grid,),
        in_specs=[
            pl.BlockSpec((_BR, _NW), lambda i: (i, 0)),
            pl.BlockSpec((_BR, d_in), lambda i: (i, 0)),
            pl.BlockSpec((d_in, hid), lambda i: (0, 0)),
        ],
        out_specs=[
            pl.BlockSpec((_BR, 1), lambda i: (i, 0)),
            pl.BlockSpec((_BR, hid), lambda i: (i, 0)),
        ],
        out_shape=[
            jax.ShapeDtypeStruct((n, 1), jnp.float32),
            jax.ShapeDtypeStruct((n_acc, hid), jnp.float32),
        ],
    )


def _mid_tc(n, n_acc, d, d_next, d_store):
    grid = n // _BR

    def body(agg_ref, g_ref, dinv_ref, b_ref, w_ref, out_ref):
        s = (agg_ref[0] + agg_ref[1] + g_ref[...]) * dinv_ref[...] + b_ref[...]
        h = jnp.maximum(s, 0.0)
        w = w_ref[...]
        if d_store > d_next:
            w = jnp.concatenate(
                [w, jnp.zeros((d, d_store - d_next), jnp.float32)], axis=1)
        out_ref[...] = jnp.dot(h, w,
                               preferred_element_type=jnp.float32,
                               precision=lax.Precision.HIGHEST) * dinv_ref[...]

    return pl.pallas_call(
        body,
        grid=(grid,),
        in_specs=[
            pl.BlockSpec((_NC, _BR, d), lambda i: (0, i, 0)),
            pl.BlockSpec((_BR, d), lambda i: (i, 0)),
            pl.BlockSpec((_BR, 1), lambda i: (i, 0)),
            pl.BlockSpec((1, d), lambda i: (0, 0)),
            pl.BlockSpec((d, d_next), lambda i: (0, 0)),
        ],
        out_specs=pl.BlockSpec((_BR, d_store), lambda i: (i, 0)),
        out_shape=jax.ShapeDtypeStruct((n_acc, d_store), jnp.float32),
    )


def _final_tc(n, d, d_store):
    grid = n // _BR

    def body(agg_ref, g_ref, dinv_ref, b_ref, out_ref):
        agg = agg_ref[0, :, :d] + agg_ref[1, :, :d] + g_ref[:, :d]
        out_ref[...] = agg * dinv_ref[...] + b_ref[...]

    return pl.pallas_call(
        body,
        grid=(grid,),
        in_specs=[
            pl.BlockSpec((_NC, _BR, d_store), lambda i: (0, i, 0)),
            pl.BlockSpec((_BR, d_store), lambda i: (i, 0)),
            pl.BlockSpec((_BR, 1), lambda i: (i, 0)),
            pl.BlockSpec((1, d), lambda i: (0, 0)),
        ],
        out_specs=pl.BlockSpec((_BR, d), lambda i: (i, 0)),
        out_shape=jax.ShapeDtypeStruct((n, d), jnp.float32),
    )


def kernel(x, edge_index, W1, b1, W2, b2, W3, b3):
    n, d_in = x.shape
    hid = W1.shape[1]
    c_out = W3.shape[1]
    e = edge_index.shape[1]

    ew = _NW * _K
    chunks = ((-(-e // ew)) + 15) // 16 * 16
    e_pad = chunks * ew
    slab = _NS * _K
    n_acc = -(-(n + 1) // slab) * slab

    pad = e_pad - e
    fill = n + jnp.arange(pad, dtype=jnp.int32) % (n_acc - n)
    srcp = jnp.concatenate([edge_index[0], fill]).reshape(_NW, chunks, _K)
    dstp = jnp.concatenate([edge_index[1], fill]).reshape(_NW, chunks, _K)

    c_store = -(-c_out // 128) * 128

    degp = _deg_kernel(n_acc, chunks)(dstp)
    dinv, g1 = _first_tc(n, n_acc, d_in, hid)(degp.T, x, W1)
    agg1 = _agg_kernel(n_acc, hid, chunks)(g1, srcp, dstp)
    g2 = _mid_tc(n, n_acc, hid, hid, hid)(agg1, g1, dinv, b1.reshape(1, -1), W2)
    agg2 = _agg_kernel(n_acc, hid, chunks)(g2, srcp, dstp)
    g3 = _mid_tc(n, n_acc, hid, c_out, c_store)(agg2, g2, dinv,
                                                b2.reshape(1, -1), W3)
    agg3 = _agg_kernel(n_acc, c_store, chunks)(g3, srcp, dstp)
    return _final_tc(n, c_out, c_store)(agg3, g3, dinv, b3.reshape(1, -1))
